# baseline (speedup 1.0000x reference)
import jax
import jax.numpy as jnp
from jax.experimental import pallas as pl
from jax.experimental.pallas import tpu as pltpu

X_ID = 343
NBUF = 8


def _copy_kernel(vl_smem, tok_ref, prompts_ref, tps_ref, tv_ref, out_hbm,
                 tmpl_v, sems):
    n = out_hbm.shape[0]
    t = tok_ref.shape[1]
    n_ctx = prompts_ref.shape[1]
    tok_row = tok_ref[...]
    xm_row = (tok_row == X_ID).astype(jnp.int32)
    r = jax.lax.broadcasted_iota(jnp.int32, (t, t), 0)
    c = jax.lax.broadcasted_iota(jnp.int32, (t, t), 1)
    cnt_incl = jnp.sum(jnp.where(c <= r, xm_row, 0), axis=1, keepdims=True)
    cnt_excl = jnp.sum(jnp.where(c < r, xm_row, 0), axis=1, keepdims=True)
    xm = (cnt_incl - cnt_excl) > 0
    cnt = cnt_incl - 1
    base = tps_ref[0]
    for j in range(n_ctx):
        base = jnp.where(xm & (cnt == j), prompts_ref[0, j][None, :], base)
    view_slot = xm & (cnt == n_ctx)
    tmpl_v[0] = jnp.where(view_slot, tv_ref[0, 1][None, :], base)
    tmpl_v[1] = jnp.where(view_slot, tv_ref[0, 2][None, :], base)

    def _dma(i):
        return pltpu.make_async_copy(
            tmpl_v.at[vl_smem[i]], out_hbm.at[i], sems.at[jax.lax.rem(i, NBUF)]
        )

    def body(i, _):
        @pl.when(i >= NBUF)
        def _():
            _dma(i - NBUF).wait()
        _dma(i).start()
        return 0

    jax.lax.fori_loop(0, n, body, 0)
    for k in range(NBUF):
        _dma(n - NBUF + k).wait()


def kernel(view_label, prompts, token_prefix_suffix, token_view, tokenized_prompts):
    b = view_label.shape[0]
    t, d = token_prefix_suffix.shape[1], token_prefix_suffix.shape[2]
    tok = tokenized_prompts.astype(jnp.int32).reshape(1, t)
    vl = view_label.astype(jnp.int32)
    grid_spec = pltpu.PrefetchScalarGridSpec(
        num_scalar_prefetch=1,
        grid=(1,),
        in_specs=[
            pl.BlockSpec((1, t), lambda i, vl_ref: (0, 0)),
            pl.BlockSpec((1, prompts.shape[1], d), lambda i, vl_ref: (0, 0, 0)),
            pl.BlockSpec((1, t, d), lambda i, vl_ref: (0, 0, 0)),
            pl.BlockSpec((1, t, d), lambda i, vl_ref: (0, 0, 0)),
        ],
        out_specs=pl.BlockSpec(memory_space=pl.ANY),
        scratch_shapes=[
            pltpu.VMEM((2, t, d), token_prefix_suffix.dtype),
            pltpu.SemaphoreType.DMA((NBUF,)),
        ],
    )
    return pl.pallas_call(
        _copy_kernel,
        grid_spec=grid_spec,
        out_shape=jax.ShapeDtypeStruct((b, t, d), token_prefix_suffix.dtype),
    )(vl, tok, prompts, token_prefix_suffix, token_view)


# speedup vs baseline: 1.2232x; 1.2232x over previous
import jax
import jax.numpy as jnp
from jax.experimental import pallas as pl
from jax.experimental.pallas import tpu as pltpu

X_ID = 343
NBUF = 8
GROUP = 4


def _copy_kernel(codes_smem, tok_ref, prompts_ref, tps_ref, tv_ref, out_hbm,
                 buf_v, sems):
    n_groups = out_hbm.shape[0] // GROUP
    t = tok_ref.shape[1]
    n_ctx = prompts_ref.shape[1]
    tok_row = tok_ref[...]
    xm_row = (tok_row == X_ID).astype(jnp.int32)
    r = jax.lax.broadcasted_iota(jnp.int32, (t, t), 0)
    c = jax.lax.broadcasted_iota(jnp.int32, (t, t), 1)
    cnt_incl = jnp.sum(jnp.where(c <= r, xm_row, 0), axis=1, keepdims=True)
    cnt_excl = jnp.sum(jnp.where(c < r, xm_row, 0), axis=1, keepdims=True)
    xm = (cnt_incl - cnt_excl) > 0
    cnt = cnt_incl - 1
    base = tps_ref[0]
    for j in range(n_ctx):
        base = jnp.where(xm & (cnt == j), prompts_ref[0, j][None, :], base)
    view_slot = xm & (cnt == n_ctx)
    tmpl0 = jnp.where(view_slot, tv_ref[0, 1][None, :], base)
    tmpl1 = jnp.where(view_slot, tv_ref[0, 2][None, :], base)
    for q in range(2 ** GROUP):
        for k in range(GROUP):
            buf_v[q, k] = tmpl1 if (q >> k) & 1 else tmpl0

    def _dma(i):
        return pltpu.make_async_copy(
            buf_v.at[codes_smem[i]],
            out_hbm.at[pl.ds(i * GROUP, GROUP)],
            sems.at[jax.lax.rem(i, NBUF)],
        )

    def body(i, _):
        @pl.when(i >= NBUF)
        def _():
            _dma(i - NBUF).wait()
        _dma(i).start()
        return 0

    jax.lax.fori_loop(0, n_groups, body, 0)
    for k in range(NBUF):
        _dma(n_groups - NBUF + k).wait()


def kernel(view_label, prompts, token_prefix_suffix, token_view, tokenized_prompts):
    b = view_label.shape[0]
    t, d = token_prefix_suffix.shape[1], token_prefix_suffix.shape[2]
    tok = tokenized_prompts.astype(jnp.int32).reshape(1, t)
    vl = view_label.astype(jnp.int32).reshape(b // GROUP, GROUP)
    codes = vl @ jnp.asarray([1 << k for k in range(GROUP)], dtype=jnp.int32)
    grid_spec = pltpu.PrefetchScalarGridSpec(
        num_scalar_prefetch=1,
        grid=(1,),
        in_specs=[
            pl.BlockSpec((1, t), lambda i, c_ref: (0, 0)),
            pl.BlockSpec((1, prompts.shape[1], d), lambda i, c_ref: (0, 0, 0)),
            pl.BlockSpec((1, t, d), lambda i, c_ref: (0, 0, 0)),
            pl.BlockSpec((1, t, d), lambda i, c_ref: (0, 0, 0)),
        ],
        out_specs=pl.BlockSpec(memory_space=pl.ANY),
        scratch_shapes=[
            pltpu.VMEM((2 ** GROUP, GROUP, t, d), token_prefix_suffix.dtype),
            pltpu.SemaphoreType.DMA((NBUF,)),
        ],
    )
    return pl.pallas_call(
        _copy_kernel,
        grid_spec=grid_spec,
        out_shape=jax.ShapeDtypeStruct((b, t, d), token_prefix_suffix.dtype),
    )(codes, tok, prompts, token_prefix_suffix, token_view)
